# Initial kernel scaffold; baseline (speedup 1.0000x reference)
import jax
import jax.numpy as jnp
from jax.experimental import pallas as pl


def kernel(f_x, v_x, quad_weights, det_A, faces, faces_to_edges, faces_to_edge_orientation):
    raise NotImplementedError("write your pallas kernel here")


# speedup vs baseline: 13.9964x; 13.9964x over previous
import functools

import jax
import jax.numpy as jnp
from jax import lax
from jax.experimental import pallas as pl
from jax.experimental.pallas import tpu as pltpu
from jax.experimental.pallas import tpu_sc as plsc

C = 500000
NV = 250000
NE = 750000

BC = 4096


def _tc_body(f_ref, v_ref, w_ref, det_ref, ori_ref, fte_ref,
             vert_ref, edge_ref, eidx_ref, face_ref):
    vw = v_ref[...] * w_ref[...]
    ix = lax.dot_general(vw, f_ref[...], (((1,), (1,)), ((), ())),
                         preferred_element_type=jnp.float32)
    ix = ix * det_ref[...]
    vert_ref[...] = ix[0:3, :]
    ev = ix[3:9, :]
    sw = jnp.concatenate(
        [ix[4:5, :], ix[3:4, :], ix[6:7, :], ix[5:6, :], ix[8:9, :], ix[7:8, :]],
        axis=0)
    o = ori_ref[...]
    oe = jnp.concatenate(
        [o[0:1, :], o[0:1, :], o[1:2, :], o[1:2, :], o[2:3, :], o[2:3, :]],
        axis=0)
    edge_ref[...] = jnp.where(oe == 1, ev, sw)
    e2 = fte_ref[...] * 2
    eidx_ref[...] = jnp.concatenate(
        [e2[0:1, :], e2[0:1, :] + 1, e2[1:2, :], e2[1:2, :] + 1,
         e2[2:3, :], e2[2:3, :] + 1], axis=0)
    face_ref[...] = ix[9:10, :]


_tc_call = pl.pallas_call(
    _tc_body,
    grid=(pl.cdiv(C, BC),),
    in_specs=[
        pl.BlockSpec((BC, 16), lambda i: (i, 0)),
        pl.BlockSpec((10, 16), lambda i: (0, 0)),
        pl.BlockSpec((1, 16), lambda i: (0, 0)),
        pl.BlockSpec((1, BC), lambda i: (0, i)),
        pl.BlockSpec((3, BC), lambda i: (0, i)),
        pl.BlockSpec((3, BC), lambda i: (0, i)),
    ],
    out_specs=[
        pl.BlockSpec((3, BC), lambda i: (0, i)),
        pl.BlockSpec((6, BC), lambda i: (0, i)),
        pl.BlockSpec((6, BC), lambda i: (0, i)),
        pl.BlockSpec((1, BC), lambda i: (0, i)),
    ],
    out_shape=[
        jax.ShapeDtypeStruct((3, C), jnp.float32),
        jax.ShapeDtypeStruct((6, C), jnp.float32),
        jax.ShapeDtypeStruct((6, C), jnp.int32),
        jax.ShapeDtypeStruct((1, C), jnp.float32),
    ],
)

NT = 16
CR = 32
ZB = 4096
OB = 8192

LV = 3 * C
ROWS_V = 11776
TR_V = ROWS_V // NT
PAD_V = ROWS_V * 128 - LV

LE = 6 * C
ROWS_E = 23552
TR_E = ROWS_E // NT
PAD_E = ROWS_E * 128 - LE

VACC_T = 15632
VACC_P = NT * VACC_T
EACC_T = 93760
EACC_P = NT * EACC_T


def _seg_scatter(vals, idx, out, acc, zb, val_b, idx_b, ob, sid,
                 rows_t, acc_t, out_total):
    base = sid * acc_t
    row0 = sid * rows_t
    nz_full, nz_tail = divmod(acc_t, ZB)

    @pl.loop(0, nz_full)
    def _(z):
        pltpu.sync_copy(zb, acc.at[pl.ds(base + z * ZB, ZB)])
    if nz_tail:
        pltpu.sync_copy(zb.at[pl.ds(0, nz_tail)],
                        acc.at[pl.ds(base + nz_full * ZB, nz_tail)])
    plsc.subcore_barrier()

    @pl.loop(0, rows_t, step=CR)
    def _(k):
        pltpu.sync_copy(vals.at[pl.ds(row0 + k, CR), :], val_b)
        pltpu.sync_copy(idx.at[pl.ds(row0 + k, CR), :], idx_b)

        @pl.loop(0, CR)
        def _(j):
            pltpu.sync_copy(val_b.at[j], acc.at[idx_b.at[j]], add=True)

    plsc.subcore_barrier()

    nfull = acc_t // OB
    tail = acc_t - nfull * OB
    tail_last = (out_total - 15 * acc_t) - nfull * OB

    @pl.loop(0, nfull)
    def _(z):
        pltpu.sync_copy(acc.at[pl.ds(base + z * OB, OB)], ob)
        pltpu.sync_copy(ob, out.at[pl.ds(base + z * OB, OB)])

    @pl.when(sid < NT - 1)
    def _():
        pltpu.sync_copy(acc.at[pl.ds(base + nfull * OB, tail)],
                        ob.at[pl.ds(0, tail)])
        pltpu.sync_copy(ob.at[pl.ds(0, tail)],
                        out.at[pl.ds(base + nfull * OB, tail)])

    @pl.when(sid == NT - 1)
    def _():
        pltpu.sync_copy(acc.at[pl.ds(base + nfull * OB, tail_last)],
                        ob.at[pl.ds(0, tail_last)])
        pltpu.sync_copy(ob.at[pl.ds(0, tail_last)],
                        out.at[pl.ds(base + nfull * OB, tail_last)])


def _sc_body(vvals, vidx, evals, eidx, zv, vout, eout,
             vacc, eacc, zb, val_b, idx_b, ob):
    core = lax.axis_index("core")
    sid = lax.axis_index("subcore")
    pltpu.sync_copy(zv, zb)

    @pl.when(core == 0)
    def _vertex():
        _seg_scatter(vvals, vidx, vout, vacc, zb, val_b, idx_b, ob, sid,
                     TR_V, VACC_T, NV)

    @pl.when(core == 1)
    def _edge():
        _seg_scatter(evals, eidx, eout, eacc, zb, val_b, idx_b, ob, sid,
                     TR_E, EACC_T, 2 * NE)


@functools.cache
def _sc_call():
    mesh = plsc.VectorSubcoreMesh(core_axis_name="core", subcore_axis_name="subcore",
                                  num_cores=2, num_subcores=NT)
    return pl.kernel(
        _sc_body,
        out_type=(
            jax.ShapeDtypeStruct((NV,), jnp.float32),
            jax.ShapeDtypeStruct((2 * NE,), jnp.float32),
        ),
        mesh=mesh,
        scratch_types=[
            pltpu.VMEM_SHARED((VACC_P,), jnp.float32),
            pltpu.VMEM_SHARED((EACC_P,), jnp.float32),
            pltpu.VMEM((ZB,), jnp.float32),
            pltpu.VMEM((CR, 128), jnp.float32),
            pltpu.VMEM((CR, 128), jnp.int32),
            pltpu.VMEM((OB,), jnp.float32),
        ],
    )


def kernel(f_x, v_x, quad_weights, det_A, faces, faces_to_edges,
           faces_to_edge_orientation):
    vert_vals, edge_vals, edge_widx, face_t = _tc_call(
        f_x, v_x, quad_weights.reshape(1, 16), det_A.reshape(1, C),
        faces_to_edge_orientation.T, faces_to_edges.T)
    face_dofs = face_t.reshape(C, 1)

    vv = jnp.pad(vert_vals.reshape(-1), (0, PAD_V)).reshape(ROWS_V, 128)
    vi = jnp.pad(faces.T.reshape(-1), (0, PAD_V)).reshape(ROWS_V, 128)
    ev = jnp.pad(edge_vals.reshape(-1), (0, PAD_E)).reshape(ROWS_E, 128)
    ei = jnp.pad(edge_widx.reshape(-1), (0, PAD_E)).reshape(ROWS_E, 128)
    zv = jnp.zeros((ZB,), jnp.float32)

    vertex_dofs, edge_flat = _sc_call()(vv, vi, ev, ei, zv)
    return (vertex_dofs, edge_flat.reshape(NE, 2), face_dofs)


# speedup vs baseline: 14.3888x; 1.0280x over previous
import functools

import jax
import jax.numpy as jnp
from jax import lax
from jax.experimental import pallas as pl
from jax.experimental.pallas import tpu as pltpu
from jax.experimental.pallas import tpu_sc as plsc

C = 500000
NV = 250000
NE = 750000

BC = 8192
CP = 507904


def _tc_body(f_ref, v_ref, w_ref, det_ref, ori_ref, fte_ref, fv_ref,
             vert_ref, edge_ref, eidx_ref, vidx_ref, face_ref):
    cell = lax.broadcasted_iota(jnp.int32, (1, BC), 1) + pl.program_id(0) * BC
    m = cell < C
    vw = v_ref[...] * w_ref[...]
    ix = lax.dot_general(vw, f_ref[...], (((1,), (1,)), ((), ())),
                         preferred_element_type=jnp.float32)
    ix = ix * det_ref[...]
    vert_ref[...] = jnp.where(m, ix[0:3, :], 0.0)
    ev = ix[3:9, :]
    sw = jnp.concatenate(
        [ix[4:5, :], ix[3:4, :], ix[6:7, :], ix[5:6, :], ix[8:9, :], ix[7:8, :]],
        axis=0)
    o = ori_ref[...]
    oe = jnp.concatenate(
        [o[0:1, :], o[0:1, :], o[1:2, :], o[1:2, :], o[2:3, :], o[2:3, :]],
        axis=0)
    edge_ref[...] = jnp.where(m & (oe == 1), ev, jnp.where(m, sw, 0.0))
    e2 = fte_ref[...] * 2
    ei = jnp.concatenate(
        [e2[0:1, :], e2[0:1, :] + 1, e2[1:2, :], e2[1:2, :] + 1,
         e2[2:3, :], e2[2:3, :] + 1], axis=0)
    eidx_ref[...] = jnp.where(m, ei, 0)
    vidx_ref[...] = jnp.where(m, fv_ref[...], 0)
    face_ref[...] = ix[9:10, :]


_tc_call = pl.pallas_call(
    _tc_body,
    grid=(CP // BC,),
    in_specs=[
        pl.BlockSpec((BC, 16), lambda i: (i, 0)),
        pl.BlockSpec((10, 16), lambda i: (0, 0)),
        pl.BlockSpec((1, 16), lambda i: (0, 0)),
        pl.BlockSpec((1, BC), lambda i: (0, i)),
        pl.BlockSpec((3, BC), lambda i: (0, i)),
        pl.BlockSpec((3, BC), lambda i: (0, i)),
        pl.BlockSpec((3, BC), lambda i: (0, i)),
    ],
    out_specs=[
        pl.BlockSpec((3, BC), lambda i: (0, i)),
        pl.BlockSpec((6, BC), lambda i: (0, i)),
        pl.BlockSpec((6, BC), lambda i: (0, i)),
        pl.BlockSpec((3, BC), lambda i: (0, i)),
        pl.BlockSpec((1, BC), lambda i: (0, i)),
    ],
    out_shape=[
        jax.ShapeDtypeStruct((3, CP), jnp.float32),
        jax.ShapeDtypeStruct((6, CP), jnp.float32),
        jax.ShapeDtypeStruct((6, CP), jnp.int32),
        jax.ShapeDtypeStruct((3, CP), jnp.int32),
        jax.ShapeDtypeStruct((1, C), jnp.float32),
    ],
)

NT = 16
CR = 24
ZB = 4096
OB = 8192

ROWS_V = 3 * CP // 128
TR_V = ROWS_V // NT
ROWS_E = 6 * CP // 128
TR_E = ROWS_E // NT

VACC_T = 15632
VACC_P = NT * VACC_T
EACC_T = 93760
EACC_P = NT * EACC_T


def _seg_scatter(vals, idx, out, acc, zb, val_b, idx_b, ob, sid,
                 rows_t, acc_t, out_total):
    base = sid * acc_t
    row0 = sid * rows_t
    nz_full, nz_tail = divmod(acc_t, ZB)

    @pl.loop(0, nz_full)
    def _(z):
        pltpu.sync_copy(zb, acc.at[pl.ds(base + z * ZB, ZB)])
    if nz_tail:
        pltpu.sync_copy(zb.at[pl.ds(0, nz_tail)],
                        acc.at[pl.ds(base + nz_full * ZB, nz_tail)])
    plsc.subcore_barrier()

    @pl.loop(0, rows_t, step=CR)
    def _(k):
        pltpu.sync_copy(vals.at[pl.ds(row0 + k, CR), :], val_b)
        pltpu.sync_copy(idx.at[pl.ds(row0 + k, CR), :], idx_b)

        @pl.loop(0, CR)
        def _(j):
            pltpu.sync_copy(val_b.at[j], acc.at[idx_b.at[j]], add=True)

    plsc.subcore_barrier()

    nfull = acc_t // OB
    tail = acc_t - nfull * OB
    tail_last = (out_total - 15 * acc_t) - nfull * OB

    @pl.loop(0, nfull)
    def _(z):
        pltpu.sync_copy(acc.at[pl.ds(base + z * OB, OB)], ob)
        pltpu.sync_copy(ob, out.at[pl.ds(base + z * OB, OB)])

    @pl.when(sid < NT - 1)
    def _():
        pltpu.sync_copy(acc.at[pl.ds(base + nfull * OB, tail)],
                        ob.at[pl.ds(0, tail)])
        pltpu.sync_copy(ob.at[pl.ds(0, tail)],
                        out.at[pl.ds(base + nfull * OB, tail)])

    @pl.when(sid == NT - 1)
    def _():
        pltpu.sync_copy(acc.at[pl.ds(base + nfull * OB, tail_last)],
                        ob.at[pl.ds(0, tail_last)])
        pltpu.sync_copy(ob.at[pl.ds(0, tail_last)],
                        out.at[pl.ds(base + nfull * OB, tail_last)])


def _sc_body(vvals, vidx, evals, eidx, zv, vout, eout,
             vacc, eacc, zb, val_b, idx_b, ob):
    core = lax.axis_index("core")
    sid = lax.axis_index("subcore")
    pltpu.sync_copy(zv, zb)

    @pl.when(core == 0)
    def _vertex():
        _seg_scatter(vvals, vidx, vout, vacc, zb, val_b, idx_b, ob, sid,
                     TR_V, VACC_T, NV)

    @pl.when(core == 1)
    def _edge():
        _seg_scatter(evals, eidx, eout, eacc, zb, val_b, idx_b, ob, sid,
                     TR_E, EACC_T, 2 * NE)


@functools.cache
def _sc_call():
    mesh = plsc.VectorSubcoreMesh(core_axis_name="core", subcore_axis_name="subcore",
                                  num_cores=2, num_subcores=NT)
    return pl.kernel(
        _sc_body,
        out_type=(
            jax.ShapeDtypeStruct((NV,), jnp.float32),
            jax.ShapeDtypeStruct((2 * NE,), jnp.float32),
        ),
        mesh=mesh,
        scratch_types=[
            pltpu.VMEM_SHARED((VACC_P,), jnp.float32),
            pltpu.VMEM_SHARED((EACC_P,), jnp.float32),
            pltpu.VMEM((ZB,), jnp.float32),
            pltpu.VMEM((CR, 128), jnp.float32),
            pltpu.VMEM((CR, 128), jnp.int32),
            pltpu.VMEM((OB,), jnp.float32),
        ],
    )


def kernel(f_x, v_x, quad_weights, det_A, faces, faces_to_edges,
           faces_to_edge_orientation):
    vert_vals, edge_vals, edge_widx, vert_idx, face_t = _tc_call(
        f_x, v_x, quad_weights.reshape(1, 16), det_A.reshape(1, C),
        faces_to_edge_orientation.T, faces_to_edges.T, faces.T)
    face_dofs = face_t.reshape(C, 1)

    vv = vert_vals.reshape(ROWS_V, 128)
    vi = vert_idx.reshape(ROWS_V, 128)
    ev = edge_vals.reshape(ROWS_E, 128)
    ei = edge_widx.reshape(ROWS_E, 128)
    zv = jnp.zeros((ZB,), jnp.float32)

    vertex_dofs, edge_flat = _sc_call()(vv, vi, ev, ei, zv)
    return (vertex_dofs, edge_flat.reshape(NE, 2), face_dofs)


# speedup vs baseline: 15.6434x; 1.0872x over previous
import functools

import jax
import jax.numpy as jnp
from jax import lax
from jax.experimental import pallas as pl
from jax.experimental.pallas import tpu as pltpu
from jax.experimental.pallas import tpu_sc as plsc

C = 500000
NV = 250000
NE = 750000

BC = 8192
CP = 507904


def _tc_body(f_ref, v_ref, w_ref, det_ref, ori_ref, fte_ref, fv_ref,
             vert_ref, edge_ref, eidx_ref, vidx_ref, face_ref):
    cell = lax.broadcasted_iota(jnp.int32, (1, BC), 1) + pl.program_id(0) * BC
    m = cell < C
    vw = v_ref[...] * w_ref[...]
    ix = lax.dot_general(vw, f_ref[...], (((1,), (1,)), ((), ())),
                         preferred_element_type=jnp.float32)
    ix = ix * det_ref[...]
    vert_ref[...] = jnp.where(m, ix[0:3, :], 0.0)
    ev = ix[3:9, :]
    sw = jnp.concatenate(
        [ix[4:5, :], ix[3:4, :], ix[6:7, :], ix[5:6, :], ix[8:9, :], ix[7:8, :]],
        axis=0)
    o = ori_ref[...]
    oe = jnp.concatenate(
        [o[0:1, :], o[0:1, :], o[1:2, :], o[1:2, :], o[2:3, :], o[2:3, :]],
        axis=0)
    edge_ref[...] = jnp.where(m & (oe == 1), ev, jnp.where(m, sw, 0.0))
    e2 = fte_ref[...] * 2
    ei = jnp.concatenate(
        [e2[0:1, :], e2[0:1, :] + 1, e2[1:2, :], e2[1:2, :] + 1,
         e2[2:3, :], e2[2:3, :] + 1], axis=0)
    eidx_ref[...] = jnp.where(m, ei, 0)
    vidx_ref[...] = jnp.where(m, fv_ref[...], 0)
    face_ref[...] = ix[9:10, :]


_tc_call = pl.pallas_call(
    _tc_body,
    grid=(CP // BC,),
    in_specs=[
        pl.BlockSpec((BC, 16), lambda i: (i, 0)),
        pl.BlockSpec((10, 16), lambda i: (0, 0)),
        pl.BlockSpec((1, 16), lambda i: (0, 0)),
        pl.BlockSpec((1, BC), lambda i: (0, i)),
        pl.BlockSpec((3, BC), lambda i: (0, i)),
        pl.BlockSpec((3, BC), lambda i: (0, i)),
        pl.BlockSpec((3, BC), lambda i: (0, i)),
    ],
    out_specs=[
        pl.BlockSpec((3, BC), lambda i: (0, i)),
        pl.BlockSpec((6, BC), lambda i: (0, i)),
        pl.BlockSpec((6, BC), lambda i: (0, i)),
        pl.BlockSpec((3, BC), lambda i: (0, i)),
        pl.BlockSpec((1, BC), lambda i: (0, i)),
    ],
    out_shape=[
        jax.ShapeDtypeStruct((3, CP), jnp.float32),
        jax.ShapeDtypeStruct((6, CP), jnp.float32),
        jax.ShapeDtypeStruct((6, CP), jnp.int32),
        jax.ShapeDtypeStruct((3, CP), jnp.int32),
        jax.ShapeDtypeStruct((1, C), jnp.float32),
    ],
)

NT = 16
CR = 24
ZB = 4096
OB = 8192

ROWS_V = 3 * CP // 128
TR_V = ROWS_V // NT
ROWS_E = 6 * CP // 128
TR_E = ROWS_E // NT

VACC_T = 15632
VACC_P = NT * VACC_T
EACC_T = 93760
EACC_P = NT * EACC_T


def _seg_scatter(vals, idx, out, acc, zb, val_b, idx_b, ob, sem, sid,
                 rows_t, acc_t, out_total):
    base = sid * acc_t
    row0 = sid * rows_t
    nz_full, nz_tail = divmod(acc_t, ZB)

    @pl.loop(0, nz_full)
    def _(z):
        pltpu.sync_copy(zb, acc.at[pl.ds(base + z * ZB, ZB)])
    if nz_tail:
        pltpu.sync_copy(zb.at[pl.ds(0, nz_tail)],
                        acc.at[pl.ds(base + nz_full * ZB, nz_tail)])
    plsc.subcore_barrier()

    @pl.loop(0, rows_t, step=CR)
    def _(k):
        pltpu.sync_copy(vals.at[pl.ds(row0 + k, CR), :], val_b)
        pltpu.sync_copy(idx.at[pl.ds(row0 + k, CR), :], idx_b)

        @pl.loop(0, CR)
        def _(j):
            pltpu.async_copy(val_b.at[j], acc.at[idx_b.at[j]], sem, add=True)

        @pl.loop(0, CR)
        def _(j):
            pltpu.make_async_copy(val_b.at[j], acc.at[idx_b.at[j]], sem).wait()

    plsc.subcore_barrier()

    nfull = acc_t // OB
    tail = acc_t - nfull * OB
    tail_last = (out_total - 15 * acc_t) - nfull * OB

    @pl.loop(0, nfull)
    def _(z):
        pltpu.sync_copy(acc.at[pl.ds(base + z * OB, OB)], ob)
        pltpu.sync_copy(ob, out.at[pl.ds(base + z * OB, OB)])

    @pl.when(sid < NT - 1)
    def _():
        pltpu.sync_copy(acc.at[pl.ds(base + nfull * OB, tail)],
                        ob.at[pl.ds(0, tail)])
        pltpu.sync_copy(ob.at[pl.ds(0, tail)],
                        out.at[pl.ds(base + nfull * OB, tail)])

    @pl.when(sid == NT - 1)
    def _():
        pltpu.sync_copy(acc.at[pl.ds(base + nfull * OB, tail_last)],
                        ob.at[pl.ds(0, tail_last)])
        pltpu.sync_copy(ob.at[pl.ds(0, tail_last)],
                        out.at[pl.ds(base + nfull * OB, tail_last)])


def _sc_body(vvals, vidx, evals, eidx, zv, vout, eout,
             vacc, eacc, zb, val_b, idx_b, ob, sem):
    core = lax.axis_index("core")
    sid = lax.axis_index("subcore")
    pltpu.sync_copy(zv, zb)

    @pl.when(core == 0)
    def _vertex():
        _seg_scatter(vvals, vidx, vout, vacc, zb, val_b, idx_b, ob, sem, sid,
                     TR_V, VACC_T, NV)

    @pl.when(core == 1)
    def _edge():
        _seg_scatter(evals, eidx, eout, eacc, zb, val_b, idx_b, ob, sem, sid,
                     TR_E, EACC_T, 2 * NE)


@functools.cache
def _sc_call():
    mesh = plsc.VectorSubcoreMesh(core_axis_name="core", subcore_axis_name="subcore",
                                  num_cores=2, num_subcores=NT)
    return pl.kernel(
        _sc_body,
        out_type=(
            jax.ShapeDtypeStruct((NV,), jnp.float32),
            jax.ShapeDtypeStruct((2 * NE,), jnp.float32),
        ),
        mesh=mesh,
        scratch_types=[
            pltpu.VMEM_SHARED((VACC_P,), jnp.float32),
            pltpu.VMEM_SHARED((EACC_P,), jnp.float32),
            pltpu.VMEM((ZB,), jnp.float32),
            pltpu.VMEM((CR, 128), jnp.float32),
            pltpu.VMEM((CR, 128), jnp.int32),
            pltpu.VMEM((OB,), jnp.float32),
            pltpu.SemaphoreType.DMA,
        ],
    )


def kernel(f_x, v_x, quad_weights, det_A, faces, faces_to_edges,
           faces_to_edge_orientation):
    vert_vals, edge_vals, edge_widx, vert_idx, face_t = _tc_call(
        f_x, v_x, quad_weights.reshape(1, 16), det_A.reshape(1, C),
        faces_to_edge_orientation.T, faces_to_edges.T, faces.T)
    face_dofs = face_t.reshape(C, 1)

    vv = vert_vals.reshape(ROWS_V, 128)
    vi = vert_idx.reshape(ROWS_V, 128)
    ev = edge_vals.reshape(ROWS_E, 128)
    ei = edge_widx.reshape(ROWS_E, 128)
    zv = jnp.zeros((ZB,), jnp.float32)

    vertex_dofs, edge_flat = _sc_call()(vv, vi, ev, ei, zv)
    return (vertex_dofs, edge_flat.reshape(NE, 2), face_dofs)
